# Initial kernel scaffold; baseline (speedup 1.0000x reference)
import jax
import jax.numpy as jnp
from jax.experimental import pallas as pl


def kernel(x, edge_index, batch, W1, b1, W2, b2, W3, b3, fc1_W, fc1_b, fc2_W, fc2_b):
    raise NotImplementedError("write your pallas kernel here")


# speedup vs baseline: 21.8894x; 21.8894x over previous
import functools

import jax
import jax.numpy as jnp
from jax import lax
from jax.experimental import pallas as pl
from jax.experimental.pallas import tpu as pltpu
from jax.experimental.pallas import tpu_sc as plsc

N = 10000
E = 320000
G = 64
DP = 128

NC, NS = 2, 16
NW = NC * NS
EPT = E // NW
NCHUNK, CHUNK = 80, 125
NBLK, BCH = 5, 16
ND = 10240
NPT = ND // NS
WBC = 128
NWB = NPT // WBC

_MESH = dict(core_axis_name="c", subcore_axis_name="s", num_cores=NC,
             num_subcores=NS)


def _make_edge_scatter():
    mesh = plsc.VectorSubcoreMesh(**_MESH)

    @functools.partial(
        pl.kernel,
        out_type=jax.ShapeDtypeStruct((NC, ND, DP), jnp.float32),
        mesh=mesh,
        scratch_types=[
            pltpu.VMEM((BCH, CHUNK), jnp.int32),
            pltpu.VMEM((BCH, CHUNK), jnp.int32),
            pltpu.VMEM((WBC, DP), jnp.float32),
            pltpu.VMEM((WBC, DP), jnp.float32),
            pltpu.VMEM_SHARED((ND, DP), jnp.float32),
            pltpu.SemaphoreType.DMA,
            pltpu.SemaphoreType.DMA,
        ],
    )
    def edge_kernel(y_hbm, src_hbm, dst_hbm, out_hbm, src_b, dst_b, r0, r1,
                    acc_sp, sem0, sem1):
        cid = lax.axis_index("c")
        sid = lax.axis_index("s")
        wid = cid * NS + sid
        zero = jnp.zeros((16,), jnp.float32)

        def zrow(i, _):
            for c in range(DP // 16):
                r0[i, pl.ds(c * 16, 16)] = zero
            return _

        lax.fori_loop(0, WBC, zrow, 0)

        def zcopy(k, _):
            pltpu.sync_copy(r0, acc_sp.at[pl.ds(sid * NPT + k * WBC, WBC)])
            return _

        lax.fori_loop(0, NWB, zcopy, 0)
        plsc.subcore_barrier()

        bufs = (r0, r1)
        sems = (sem0, sem1)

        def blk(b, _):
            pltpu.sync_copy(src_hbm.at[wid, pl.ds(b * BCH, BCH)], src_b)
            pltpu.sync_copy(dst_hbm.at[wid, pl.ds(b * BCH, BCH)], dst_b)
            pltpu.async_copy(y_hbm.at[src_b.at[0]],
                             r0.at[pl.ds(0, CHUNK)], sem0)

            def step(s, _2):
                for p in range(2):
                    c = 2 * s + p
                    buf, sem = bufs[p], sems[p]
                    obuf, osem = bufs[1 - p], sems[1 - p]
                    pltpu.make_async_copy(y_hbm.at[src_b.at[c]],
                                          buf.at[pl.ds(0, CHUNK)], sem).wait()

                    @pl.when(c + 1 < BCH)
                    def _3():
                        pltpu.async_copy(y_hbm.at[src_b.at[c + 1]],
                                         obuf.at[pl.ds(0, CHUNK)], osem)

                    pltpu.sync_copy(buf.at[pl.ds(0, CHUNK)],
                                    acc_sp.at[dst_b.at[c]], add=True)
                return _2

            lax.fori_loop(0, BCH // 2, step, 0)
            return _

        lax.fori_loop(0, NBLK, blk, 0)
        plsc.subcore_barrier()

        def wb(k, _):
            pltpu.sync_copy(acc_sp.at[pl.ds(sid * NPT + k * WBC, WBC)],
                            r0)
            pltpu.sync_copy(r0,
                            out_hbm.at[cid, pl.ds(sid * NPT + k * WBC, WBC)])
            return _

        lax.fori_loop(0, NWB, wb, 0)

    return edge_kernel


def _make_deg_kernel():
    mesh = plsc.VectorSubcoreMesh(**_MESH)

    @functools.partial(
        pl.kernel,
        out_type=jax.ShapeDtypeStruct((NC, ND, DP), jnp.float32),
        mesh=mesh,
        scratch_types=[
            pltpu.VMEM((NCHUNK, CHUNK), jnp.int32),
            pltpu.VMEM((WBC, DP), jnp.float32),
            pltpu.VMEM_SHARED((ND, DP), jnp.float32),
        ],
    )
    def deg_kernel(dst_hbm, out_hbm, dst_v, buf_v, acc_sp):
        cid = lax.axis_index("c")
        sid = lax.axis_index("s")
        wid = cid * NS + sid
        pltpu.sync_copy(dst_hbm.at[wid], dst_v)
        zero = jnp.zeros((16,), jnp.float32)
        one = jnp.full((16,), 1.0, jnp.float32)

        def zrow(i, _):
            for c in range(DP // 16):
                buf_v[i, pl.ds(c * 16, 16)] = zero
            return _

        lax.fori_loop(0, WBC, zrow, 0)

        def zcopy(k, _):
            pltpu.sync_copy(buf_v,
                            acc_sp.at[pl.ds(sid * NPT + k * WBC, WBC)])
            return _

        lax.fori_loop(0, NWB, zcopy, 0)

        def orow(i, _):
            buf_v[i, pl.ds(0, 16)] = one
            return _

        lax.fori_loop(0, CHUNK, orow, 0)
        plsc.subcore_barrier()

        def chunk(j, _):
            pltpu.sync_copy(buf_v.at[pl.ds(0, CHUNK)],
                            acc_sp.at[dst_v.at[j]], add=True)
            return _

        lax.fori_loop(0, NCHUNK, chunk, 0)
        plsc.subcore_barrier()

        def wb(k, _):
            pltpu.sync_copy(acc_sp.at[pl.ds(sid * NPT + k * WBC, WBC)],
                            buf_v)
            pltpu.sync_copy(buf_v,
                            out_hbm.at[cid, pl.ds(sid * NPT + k * WBC, WBC)])
            return _

        lax.fori_loop(0, NWB, wb, 0)

    return deg_kernel


_deg_call = _make_deg_kernel()


_edge_call = _make_edge_scatter()


def _pad_cols(a):
    return jnp.concatenate(
        [a, jnp.zeros((a.shape[0], DP - a.shape[1]), a.dtype)], axis=1)


def _tc_prep(deg_parts, x, W1):
    def body(degp_ref, x_ref, w_ref, dinv_ref, y_ref):
        degp = degp_ref[...]
        deg = 1.0 + degp[0, :N, 0] + degp[1, :N, 0]
        dinv = (1.0 / jnp.sqrt(deg))[:, None]
        dinv_ref[...] = dinv
        xw = jnp.dot(x_ref[...], w_ref[...],
                     preferred_element_type=jnp.float32)
        y_ref[...] = _pad_cols(xw * dinv)

    return pl.pallas_call(
        body,
        out_shape=(jax.ShapeDtypeStruct((N, 1), jnp.float32),
                   jax.ShapeDtypeStruct((N, DP), jnp.float32)),
    )(deg_parts, x, W1)


def _tc_mid(S_parts, y, dinv, b, W):
    df = W.shape[0]

    def body(s_ref, y_ref, dinv_ref, b_ref, w_ref, out_ref):
        s = s_ref[...]
        dinv = dinv_ref[...]
        t = (s[0, :N, :df] + s[1, :N, :df] + y_ref[:, :df])
        h = jax.nn.relu(dinv * t + b_ref[...])
        out_ref[...] = _pad_cols(
            jnp.dot(h, w_ref[...], preferred_element_type=jnp.float32) * dinv)

    return pl.pallas_call(
        body,
        out_shape=jax.ShapeDtypeStruct((N, DP), jnp.float32),
    )(S_parts, y, dinv, b, W)


def _tc_final(S_parts, y, dinv, b, batch2, fc1_W, fc1_b, fc2_W, fc2_b):
    df = fc1_W.shape[0]

    def body(s_ref, y_ref, dinv_ref, b_ref, batch_ref, w1_ref, b1_ref,
             w2_ref, b2_ref, out_ref):
        s = s_ref[...]
        t = (s[0, :N, :df] + s[1, :N, :df] + y_ref[:, :df])
        h = jax.nn.relu(dinv_ref[...] * t + b_ref[...])
        gids = lax.broadcasted_iota(jnp.int32, (N, G), 1)
        onehot = (batch_ref[...] == gids).astype(jnp.float32)
        sums = lax.dot_general(onehot, h, (((0,), (0,)), ((), ())),
                               preferred_element_type=jnp.float32,
                               precision=lax.Precision.HIGHEST)
        cnts = jnp.sum(onehot, axis=0)[:, None]
        pooled = sums / jnp.maximum(cnts, 1.0)
        hh = jax.nn.relu(jnp.dot(pooled, w1_ref[...],
                                 preferred_element_type=jnp.float32)
                         + b1_ref[...])
        out_ref[...] = jnp.dot(hh, w2_ref[...],
                               preferred_element_type=jnp.float32) + b2_ref[...]

    return pl.pallas_call(
        body,
        out_shape=jax.ShapeDtypeStruct((G, 1), jnp.float32),
    )(S_parts, y, dinv, b, batch2, fc1_W, fc1_b, fc2_W, fc2_b)


def kernel(x, edge_index, batch, W1, b1, W2, b2, W3, b3, fc1_W, fc1_b,
           fc2_W, fc2_b):
    src = edge_index[0].reshape(NW, NCHUNK, CHUNK)
    dst = edge_index[1].reshape(NW, NCHUNK, CHUNK)

    deg_parts = _deg_call(dst)
    dinv, y1 = _tc_prep(deg_parts, x, W1)

    S1 = _edge_call(y1, src, dst)
    y2 = _tc_mid(S1, y1, dinv, b1.reshape(1, -1), W2)

    S2 = _edge_call(y2, src, dst)
    y3 = _tc_mid(S2, y2, dinv, b2.reshape(1, -1), W3)

    S3 = _edge_call(y3, src, dst)
    return _tc_final(S3, y3, dinv, b3.reshape(1, -1), batch.reshape(-1, 1),
                     fc1_W, fc1_b.reshape(1, -1), fc2_W, fc2_b.reshape(1, -1))
